# baseline (speedup 1.0000x reference)
import functools

import jax
import jax.numpy as jnp
from jax import lax
from jax.experimental import pallas as pl
from jax.experimental.pallas import tpu as pltpu
from jax.experimental.pallas import tpu_sc as plsc

_B = 16384
_D = 32

_info = plsc.get_sparse_core_info()
_NC = _info.num_cores
_NS = _info.num_subcores
_NW = _NC * _NS
_BPW = _B // _NW
_CH = 128
_NCH = _BPW // _CH


@functools.partial(
    pl.kernel,
    out_type=(
        jax.ShapeDtypeStruct((_B, _D), jnp.int32),
        jax.ShapeDtypeStruct((_B, _D), jnp.float32),
    ),
    mesh=plsc.VectorSubcoreMesh(core_axis_name="c", subcore_axis_name="s"),
    scratch_types=[
        pltpu.VMEM((_BPW,), jnp.int32),
        pltpu.VMEM((_BPW, _D), jnp.int32),
        pltpu.VMEM((_BPW, _D), jnp.float32),
        pltpu.SemaphoreType.DMA((_NCH, 2)),
        pltpu.SemaphoreType.DMA,
    ],
    compiler_params=pltpu.CompilerParams(use_tc_tiling_on_sc=False),
)
def _gather2(idx_hbm, tgt_hbm, w_hbm, out_t, out_w,
             idx_v, t_rows, w_rows, sem_in, sem_out):
    wid = lax.axis_index("s") * _NC + lax.axis_index("c")
    base = wid * _BPW
    pltpu.sync_copy(idx_hbm.at[pl.ds(base, _BPW)], idx_v)
    gathers = []
    for c in range(_NCH):
        sl = pl.ds(c * _CH, _CH)
        gathers.append((
            pltpu.async_copy(tgt_hbm.at[idx_v.at[sl]], t_rows.at[sl],
                             sem_in.at[c, 0]),
            pltpu.async_copy(w_hbm.at[idx_v.at[sl]], w_rows.at[sl],
                             sem_in.at[c, 1]),
        ))
    outs = []
    for c in range(_NCH):
        sl = pl.ds(c * _CH, _CH)
        gt, gw = gathers[c]
        gt.wait()
        outs.append(pltpu.async_copy(
            t_rows.at[sl], out_t.at[pl.ds(base + c * _CH, _CH)], sem_out))
        gw.wait()
        outs.append(pltpu.async_copy(
            w_rows.at[sl], out_w.at[pl.ds(base + c * _CH, _CH)], sem_out))
    for o in outs:
        o.wait()


def kernel(src_neurons, cached_targets, weights):
    return _gather2(src_neurons.astype(jnp.int32), cached_targets, weights)


# speedup vs baseline: 1.3781x; 1.3781x over previous
import functools

import jax
import jax.numpy as jnp
from jax import lax
from jax.experimental import pallas as pl
from jax.experimental.pallas import tpu as pltpu
from jax.experimental.pallas import tpu_sc as plsc

_B = 16384
_D = 32
_NSRC = 10000
_L = 16

_NQ = 4
_BQ = _B // _NQ
_NR = _D // 8


@functools.partial(
    pl.kernel,
    out_type=(
        jax.ShapeDtypeStruct((_D, _B), jnp.int32),
        jax.ShapeDtypeStruct((_D, _B), jnp.int32),
    ),
    mesh=plsc.VectorSubcoreMesh(core_axis_name="c", subcore_axis_name="s"),
    scratch_types=[
        pltpu.VMEM((_BQ,), jnp.int32),
        pltpu.VMEM((8, _NSRC), jnp.int32),
        pltpu.VMEM((8, _BQ), jnp.int32),
    ],
    compiler_params=pltpu.CompilerParams(use_tc_tiling_on_sc=True,
                                         needs_layout_passes=False),
)
def _gather2(idx_hbm, tgtT_hbm, wT_hbm, out_t, out_w, idx_v, slab_v, res_v):
    w = lax.axis_index("s") * 2 + lax.axis_index("c")
    t = w // 16
    r = (w // 4) % _NR
    q = w % _NQ

    pltpu.sync_copy(idx_hbm.at[pl.ds(q * _BQ, _BQ)], idx_v)

    @pl.when(t == 0)
    def _load_t():
        pltpu.sync_copy(tgtT_hbm.at[pl.ds(8 * r, 8)], slab_v)

    @pl.when(t == 1)
    def _load_w():
        pltpu.sync_copy(wT_hbm.at[pl.ds(8 * r, 8)], slab_v)

    def body(i, carry):
        iv = idx_v[pl.ds(i * _L, _L)]
        for j in range(8):
            jv = jnp.full((_L,), j, jnp.int32)
            res_v[j, pl.ds(i * _L, _L)] = plsc.load_gather(slab_v, [jv, iv])
        return carry

    lax.fori_loop(0, _BQ // _L, body, 0)

    @pl.when(t == 0)
    def _store_t():
        pltpu.sync_copy(res_v, out_t.at[pl.ds(8 * r, 8), pl.ds(q * _BQ, _BQ)])

    @pl.when(t == 1)
    def _store_w():
        pltpu.sync_copy(res_v, out_w.at[pl.ds(8 * r, 8), pl.ds(q * _BQ, _BQ)])


def kernel(src_neurons, cached_targets, weights):
    wT_i32 = lax.bitcast_convert_type(weights.T, jnp.int32)
    outTt, outTw = _gather2(src_neurons.astype(jnp.int32),
                            cached_targets.T, wT_i32)
    return outTt.T, lax.bitcast_convert_type(outTw, jnp.float32).T


# speedup vs baseline: 1.7444x; 1.2659x over previous
import functools

import jax
import jax.numpy as jnp
from jax import lax
from jax.experimental import pallas as pl
from jax.experimental.pallas import tpu as pltpu
from jax.experimental.pallas import tpu_sc as plsc

_B = 16384
_D = 32
_NSRC = 10000
_L = 16

_NQ = 4
_BQ = _B // _NQ
_NR = _D // 8


@functools.partial(
    pl.kernel,
    out_type=(
        jax.ShapeDtypeStruct((_D, _B), jnp.int32),
        jax.ShapeDtypeStruct((_D, _B), jnp.int32),
    ),
    mesh=plsc.VectorSubcoreMesh(core_axis_name="c", subcore_axis_name="s"),
    scratch_types=[
        pltpu.VMEM((_BQ,), jnp.int32),
        pltpu.VMEM((8, _NSRC), jnp.int32),
        pltpu.VMEM((8, _BQ), jnp.int32),
    ],
    compiler_params=pltpu.CompilerParams(use_tc_tiling_on_sc=True,
                                         needs_layout_passes=False),
)
def _gather2(idx_hbm, tgtT_hbm, wT_hbm, out_t, out_w, idx_v, slab_v, res_v):
    w = lax.axis_index("s") * 2 + lax.axis_index("c")
    t = w // 16
    r = (w // 4) % _NR
    q = w % _NQ

    pltpu.sync_copy(idx_hbm.at[pl.ds(q * _BQ, _BQ)], idx_v)

    @pl.when(t == 0)
    def _load_t():
        pltpu.sync_copy(tgtT_hbm.at[pl.ds(8 * r, 8)], slab_v)

    @pl.when(t == 1)
    def _load_w():
        pltpu.sync_copy(wT_hbm.at[pl.ds(8 * r, 8)], slab_v)

    @plsc.parallel_loop(0, _BQ // _L, unroll=4)
    def _gather_loop(i):
        iv = idx_v[pl.ds(i * _L, _L)]
        for j in range(8):
            jv = jnp.full((_L,), j, jnp.int32)
            res_v[j, pl.ds(i * _L, _L)] = plsc.load_gather(slab_v, [jv, iv])

    @pl.when(t == 0)
    def _store_t():
        pltpu.sync_copy(res_v, out_t.at[pl.ds(8 * r, 8), pl.ds(q * _BQ, _BQ)])

    @pl.when(t == 1)
    def _store_w():
        pltpu.sync_copy(res_v, out_w.at[pl.ds(8 * r, 8), pl.ds(q * _BQ, _BQ)])


def kernel(src_neurons, cached_targets, weights):
    wT_i32 = lax.bitcast_convert_type(weights.T, jnp.int32)
    outTt, outTw = _gather2(src_neurons.astype(jnp.int32),
                            cached_targets.T, wT_i32)
    return outTt.T, lax.bitcast_convert_type(outTw, jnp.float32).T
